# baseline (speedup 1.0000x reference)
import functools

import jax
import jax.numpy as jnp
from jax import lax
from jax.experimental import pallas as pl
from jax.experimental.pallas import tpu as pltpu
from jax.experimental.pallas import tpu_sc as plsc

N_NODES = 10000
N_EDGES = 320000
D_FEAT = 128

NC = 2
NS = 16
NW = NC * NS

EDGES_PER_WORKER = N_EDGES // NW
CHUNK = 80
NCHUNK = EDGES_PER_WORKER // CHUNK
NROW = 4
NIDX = 8
LOOK = 2

STRIPE = 624
ZROWS = 8

_mesh = plsc.VectorSubcoreMesh(core_axis_name="c", subcore_axis_name="s")


@functools.partial(
    pl.kernel,
    out_type=jax.ShapeDtypeStruct((NC, N_NODES, D_FEAT), jnp.float32),
    mesh=_mesh,
    scratch_types=[
        pltpu.VMEM((NIDX, CHUNK), jnp.int32),
        pltpu.VMEM((NIDX, CHUNK), jnp.int32),
        pltpu.VMEM((NROW, CHUNK, D_FEAT), jnp.float32),
        pltpu.VMEM((ZROWS, D_FEAT), jnp.float32),
        pltpu.VMEM_SHARED((N_NODES, D_FEAT), jnp.float32),
        pltpu.SemaphoreType.DMA((NIDX,)),
        pltpu.SemaphoreType.DMA((NROW,)),
        pltpu.SemaphoreType.DMA((NROW,)),
    ],
)
def _sc_message_passing(x_hbm, ei_hbm, out_hbm,
                        sidx, didx, rows, zbuf, acc, semi, semg, sems):
    cid = lax.axis_index("c")
    sid = lax.axis_index("s")
    wid = sid * NC + cid
    base_w = wid * EDGES_PER_WORKER

    z16 = jnp.zeros((16,), jnp.float32)

    @pl.loop(0, ZROWS)
    def _(i):
        for j in range(D_FEAT // 16):
            zbuf[i, pl.ds(j * 16, 16)] = z16

    @pl.loop(0, STRIPE // ZROWS)
    def _(t):
        pltpu.async_copy(zbuf, acc.at[pl.ds(sid * STRIPE + t * ZROWS, ZROWS)], semg.at[0])

    @pl.when(sid == NS - 1)
    def _():
        for t in range((N_NODES - NS * STRIPE) // ZROWS):
            pltpu.async_copy(zbuf, acc.at[pl.ds(NS * STRIPE + t * ZROWS, ZROWS)], semg.at[0])

    @pl.loop(0, STRIPE // ZROWS)
    def _(t):
        pltpu.make_async_copy(zbuf, acc.at[pl.ds(0, ZROWS)], semg.at[0]).wait()

    @pl.when(sid == NS - 1)
    def _():
        for t in range((N_NODES - NS * STRIPE) // ZROWS):
            pltpu.make_async_copy(zbuf, acc.at[pl.ds(0, ZROWS)], semg.at[0]).wait()

    plsc.subcore_barrier()

    def idx_issue(k, si):
        b = base_w + k * CHUNK
        pltpu.async_copy(ei_hbm.at[pl.ds(b, CHUNK)], sidx.at[si], semi.at[si])
        pltpu.async_copy(ei_hbm.at[pl.ds(N_EDGES + b, CHUNK)], didx.at[si], semi.at[si])

    def idx_wait(si):
        pltpu.make_async_copy(ei_hbm.at[pl.ds(0, CHUNK)], sidx.at[si], semi.at[si]).wait()
        pltpu.make_async_copy(ei_hbm.at[pl.ds(0, CHUNK)], didx.at[si], semi.at[si]).wait()

    def g_issue(rp, si):
        pltpu.async_copy(x_hbm.at[sidx.at[si]], rows.at[rp], semg.at[rp])

    def g_wait(rp, si):
        pltpu.make_async_copy(x_hbm.at[sidx.at[si]], rows.at[rp], semg.at[rp]).wait()

    def s_issue(rp, si):
        pltpu.async_copy(rows.at[rp], acc.at[didx.at[si]], sems.at[rp], add=True)

    def s_wait(rp, si):
        pltpu.make_async_copy(rows.at[rp], acc.at[didx.at[si]], sems.at[rp]).wait()

    for k in range(2 * LOOK):
        idx_issue(k, k)
    for k in range(LOOK):
        idx_wait(k % NIDX)
        g_issue(k % NROW, k % NIDX)

    def body(k, rp, rg, sik, sig, sii, first, more_g=True, more_i=True):
        g_wait(rp, sik)
        s_issue(rp, sik)
        if not first:
            s_wait(rg, (sig - 2 * LOOK) % NIDX)
        if more_g:
            idx_wait(sig)
            g_issue(rg, sig)
        if more_i:
            @pl.when(k + 2 * LOOK < NCHUNK)
            def _():
                idx_issue(k + 2 * LOOK, sii)

    for k in range(LOOK):
        body(k, k % NROW, (k + LOOK) % NROW,
             k % NIDX, (k + LOOK) % NIDX, (k + 2 * LOOK) % NIDX, first=True)

    _EPI = (NCHUNK - LOOK) % NIDX

    @pl.loop(0, (NCHUNK - LOOK - _EPI) // NIDX)
    def _(i):
        base_k = LOOK + i * NIDX
        for j in range(NIDX):
            body(base_k + j, (LOOK + j) % NROW, j % NROW,
                 (LOOK + j) % NIDX, (2 * LOOK + j) % NIDX,
                 (3 * LOOK + j) % NIDX, first=False)

    for k in range(NCHUNK - _EPI, NCHUNK):
        body(k, k % NROW, (k + LOOK) % NROW,
             k % NIDX, (k + LOOK) % NIDX, (k + 2 * LOOK) % NIDX,
             first=False, more_g=(k + LOOK < NCHUNK), more_i=False)
    for k in range(NCHUNK - LOOK, NCHUNK):
        s_wait(k % NROW, k % NIDX)

    plsc.subcore_barrier()

    pltpu.sync_copy(
        acc.at[pl.ds(sid * STRIPE, STRIPE)],
        out_hbm.at[cid, pl.ds(sid * STRIPE, STRIPE)],
    )

    @pl.when(sid == NS - 1)
    def _():
        pltpu.sync_copy(
            acc.at[pl.ds(NS * STRIPE, N_NODES - NS * STRIPE)],
            out_hbm.at[cid, pl.ds(NS * STRIPE, N_NODES - NS * STRIPE)],
        )


def _tc_add_body(p_ref, o_ref):
    o_ref[...] = p_ref[0] + p_ref[1]


_ROWS_PER_BLOCK = 2000


def _tc_add(partials):
    return pl.pallas_call(
        _tc_add_body,
        out_shape=jax.ShapeDtypeStruct((N_NODES, D_FEAT), jnp.float32),
        grid=(N_NODES // _ROWS_PER_BLOCK,),
        in_specs=[pl.BlockSpec((NC, _ROWS_PER_BLOCK, D_FEAT), lambda i: (0, i, 0))],
        out_specs=pl.BlockSpec((_ROWS_PER_BLOCK, D_FEAT), lambda i: (i, 0)),
    )(partials)


def kernel(x, edge_index):
    ei = edge_index.astype(jnp.int32).reshape(-1)
    partials = _sc_message_passing(x, ei)
    return _tc_add(partials)


# speedup vs baseline: 1.0041x; 1.0041x over previous
import functools

import jax
import jax.numpy as jnp
from jax import lax
from jax.experimental import pallas as pl
from jax.experimental.pallas import tpu as pltpu
from jax.experimental.pallas import tpu_sc as plsc

N_NODES = 10000
N_EDGES = 320000
D_FEAT = 128

NC = 2
NS = 16
NW = NC * NS

EDGES_PER_WORKER = N_EDGES // NW
CHUNK = 80
NCHUNK = EDGES_PER_WORKER // CHUNK
NROW = 4
NIDX = 8
LOOK = 2

STRIPE = 624
ZROWS = 8

_mesh = plsc.VectorSubcoreMesh(core_axis_name="c", subcore_axis_name="s")


@functools.partial(
    pl.kernel,
    out_type=jax.ShapeDtypeStruct((NC, N_NODES, D_FEAT), jnp.float32),
    mesh=_mesh,
    scratch_types=[
        pltpu.VMEM((NIDX, CHUNK), jnp.int32),
        pltpu.VMEM((NIDX, CHUNK), jnp.int32),
        pltpu.VMEM((NROW, CHUNK, D_FEAT), jnp.float32),
        pltpu.VMEM((ZROWS, D_FEAT), jnp.float32),
        pltpu.VMEM_SHARED((N_NODES, D_FEAT), jnp.float32),
        pltpu.SemaphoreType.DMA((NIDX,)),
        pltpu.SemaphoreType.DMA((NROW,)),
        pltpu.SemaphoreType.DMA((NROW,)),
    ],
)
def _sc_message_passing(x_hbm, ei_hbm, out_hbm,
                        sidx, didx, rows, zbuf, acc, semi, semg, sems):
    cid = lax.axis_index("c")
    sid = lax.axis_index("s")
    wid = sid * NC + cid
    base_w = wid * EDGES_PER_WORKER

    z16 = jnp.zeros((16,), jnp.float32)

    @pl.loop(0, ZROWS)
    def _(i):
        for j in range(D_FEAT // 16):
            zbuf[i, pl.ds(j * 16, 16)] = z16

    @pl.loop(0, STRIPE // ZROWS)
    def _(t):
        pltpu.async_copy(zbuf, acc.at[pl.ds(sid * STRIPE + t * ZROWS, ZROWS)], semg.at[0])

    @pl.when(sid == NS - 1)
    def _():
        for t in range((N_NODES - NS * STRIPE) // ZROWS):
            pltpu.async_copy(zbuf, acc.at[pl.ds(NS * STRIPE + t * ZROWS, ZROWS)], semg.at[0])

    @pl.loop(0, STRIPE // ZROWS)
    def _(t):
        pltpu.make_async_copy(zbuf, acc.at[pl.ds(0, ZROWS)], semg.at[0]).wait()

    @pl.when(sid == NS - 1)
    def _():
        for t in range((N_NODES - NS * STRIPE) // ZROWS):
            pltpu.make_async_copy(zbuf, acc.at[pl.ds(0, ZROWS)], semg.at[0]).wait()

    plsc.subcore_barrier()

    def idx_issue(k, si):
        b = base_w + k * CHUNK
        pltpu.async_copy(ei_hbm.at[pl.ds(b, CHUNK)], sidx.at[si], semi.at[si])
        pltpu.async_copy(ei_hbm.at[pl.ds(N_EDGES + b, CHUNK)], didx.at[si], semi.at[si])

    def idx_wait(si):
        pltpu.make_async_copy(ei_hbm.at[pl.ds(0, CHUNK)], sidx.at[si], semi.at[si]).wait()
        pltpu.make_async_copy(ei_hbm.at[pl.ds(0, CHUNK)], didx.at[si], semi.at[si]).wait()

    def g_issue(rp, si):
        pltpu.async_copy(x_hbm.at[sidx.at[si]], rows.at[rp], semg.at[rp])

    def g_wait(rp, si):
        pltpu.make_async_copy(x_hbm.at[sidx.at[si]], rows.at[rp], semg.at[rp]).wait()

    def s_issue(rp, si):
        pltpu.async_copy(rows.at[rp], acc.at[didx.at[si]], sems.at[rp], add=True)

    def s_wait(rp, si):
        pltpu.make_async_copy(rows.at[rp], acc.at[didx.at[si]], sems.at[rp]).wait()

    for k in range(2 * LOOK):
        idx_issue(k, k)
    for k in range(LOOK):
        idx_wait(k % NIDX)
        g_issue(k % NROW, k % NIDX)

    def body(k, rp, rg, sik, sig, sii, first, more_g=True, more_i=True):
        g_wait(rp, sik)
        s_issue(rp, sik)
        if not first:
            s_wait(rg, (sig - 2 * LOOK) % NIDX)
        if more_g:
            idx_wait(sig)
            g_issue(rg, sig)
        if more_i:
            @pl.when(k + 2 * LOOK < NCHUNK)
            def _():
                idx_issue(k + 2 * LOOK, sii)

    for k in range(LOOK):
        body(k, k % NROW, (k + LOOK) % NROW,
             k % NIDX, (k + LOOK) % NIDX, (k + 2 * LOOK) % NIDX, first=True)

    _EPI = (NCHUNK - LOOK) % NIDX

    @pl.loop(0, (NCHUNK - LOOK - _EPI) // NIDX)
    def _(i):
        base_k = LOOK + i * NIDX
        for j in range(NIDX):
            body(base_k + j, (LOOK + j) % NROW, j % NROW,
                 (LOOK + j) % NIDX, (2 * LOOK + j) % NIDX,
                 (3 * LOOK + j) % NIDX, first=False)

    for k in range(NCHUNK - _EPI, NCHUNK):
        body(k, k % NROW, (k + LOOK) % NROW,
             k % NIDX, (k + LOOK) % NIDX, (k + 2 * LOOK) % NIDX,
             first=False, more_g=(k + LOOK < NCHUNK), more_i=False)
    for k in range(NCHUNK - LOOK, NCHUNK):
        s_wait(k % NROW, k % NIDX)

    plsc.subcore_barrier()

    pltpu.sync_copy(
        acc.at[pl.ds(sid * STRIPE, STRIPE)],
        out_hbm.at[cid, pl.ds(sid * STRIPE, STRIPE)],
    )

    @pl.when(sid == NS - 1)
    def _():
        pltpu.sync_copy(
            acc.at[pl.ds(NS * STRIPE, N_NODES - NS * STRIPE)],
            out_hbm.at[cid, pl.ds(NS * STRIPE, N_NODES - NS * STRIPE)],
        )


def _tc_add_body(p_ref, o_ref):
    o_ref[...] = p_ref[0] + p_ref[1]


_ROWS_PER_BLOCK = 10000


def _tc_add(partials):
    return pl.pallas_call(
        _tc_add_body,
        out_shape=jax.ShapeDtypeStruct((N_NODES, D_FEAT), jnp.float32),
        grid=(N_NODES // _ROWS_PER_BLOCK,),
        in_specs=[pl.BlockSpec((NC, _ROWS_PER_BLOCK, D_FEAT), lambda i: (0, i, 0))],
        out_specs=pl.BlockSpec((_ROWS_PER_BLOCK, D_FEAT), lambda i: (i, 0)),
    )(partials)


def kernel(x, edge_index):
    ei = edge_index.astype(jnp.int32).reshape(-1)
    partials = _sc_message_passing(x, ei)
    return _tc_add(partials)
